# baseline (speedup 1.0000x reference)
import jax
import jax.numpy as jnp
from jax.experimental import pallas as pl
from jax.experimental.pallas import tpu as pltpu

_N = 20000


def _tc_body(pt_ref, tt_ref, out_ref):
    P = pt_ref[...]
    T = tt_ref[...]
    AD = jnp.abs(P - T)

    p2 = pt_ref[2:3, :]
    p3 = pt_ref[3:4, :]
    t2 = tt_ref[2:3, :]
    t4 = tt_ref[4:5, :]

    e = jnp.abs(p2 - p3) > 0.5
    ew = jnp.where(e, 1.0, 0.0)
    cw = 1.0 - ew

    e_sum = jnp.sum(ew * AD)
    c_row = (AD[0:1, :] + AD[1:2, :]
             + jnp.abs(p2 + p3 - 2.0 * t2) + jnp.abs(t4))
    c_sum = jnp.sum(cw * c_row)
    ne = jnp.sum(ew)
    nc = jnp.float32(_N) - ne

    res = (e_sum / jnp.maximum(ne, 1.0) + c_sum / jnp.maximum(nc, 1.0))
    out_ref[...] = jnp.full((1, 1), res, jnp.float32)


@jax.jit
def tc_loss(pred, target):
    out = pl.pallas_call(
        _tc_body,
        out_shape=jax.ShapeDtypeStruct((1, 1), jnp.float32),
        in_specs=[pl.BlockSpec(memory_space=pltpu.VMEM),
                  pl.BlockSpec(memory_space=pltpu.VMEM)],
        out_specs=pl.BlockSpec(memory_space=pltpu.VMEM),
    )(pred.T, target.T)
    return out[0, 0]


def kernel(pred, target, cls):
    return tc_loss(pred, target)
